# baseline (speedup 1.0000x reference)
import functools

import numpy as np
import jax
import jax.numpy as jnp
from jax import lax
from jax.experimental import pallas as pl
from jax.experimental.pallas import tpu as pltpu
from jax.experimental.pallas import tpu_sc as plsc

_ALPHA = 0.1
_THETA = 0.5
_LAYERS = 4
_EPS = 1e-5
_N, _E, _D = 10000, 160000, 256
_H = _D // 2
_NS = 16
_EPT = _E // _NS
_K = 80
_NCHUNK = _EPT // _K
_RPT = 624
_TAIL = _N - _RPT * _NS

_HIGH = jax.lax.Precision.HIGHEST
_GDN = jax.lax.GatherDimensionNumbers(
    offset_dims=(), collapsed_slice_dims=(0,), start_index_map=(0,))


def _dot(a, b):
    return jax.lax.dot_general(a, b, (((1,), (0,)), ((), ())),
                               precision=_HIGH,
                               preferred_element_type=jnp.float32)


@functools.partial(
    pl.kernel,
    out_type=[jax.ShapeDtypeStruct((_N, _H), jnp.float32),
              jax.ShapeDtypeStruct((_N, _H), jnp.float32)],
    mesh=plsc.VectorSubcoreMesh(core_axis_name="c", subcore_axis_name="s"),
    scratch_types=[
        [pltpu.VMEM((_K, _H), jnp.float32)] * 3,
        [pltpu.VMEM((_K,), jnp.int32)] * 3,
        [pltpu.VMEM((_K,), jnp.int32)] * 3,
        [pltpu.VMEM((_K,), jnp.float32)] * 3,
        pltpu.VMEM_SHARED((_N, _H), jnp.float32),
        [pltpu.SemaphoreType.DMA] * 3,
        [pltpu.SemaphoreType.DMA] * 3,
        [pltpu.SemaphoreType.DMA] * 3,
    ],
)
def _sc_agg(h_lo, h_hi, src, dst, attr, out_lo, out_hi,
            msg, sb, db, ab, acc, gsem, ssem, isem):
    c = lax.axis_index("c")
    s = lax.axis_index("s")

    def run(h_tbl, out_tbl):
        ebase = pl.multiple_of(s * _EPT, 8)

        def load_idx(i, b):
            pltpu.async_copy(src.at[pl.ds(ebase + i * _K, _K)], sb[b],
                             isem[b])

        def start(i, b):
            pltpu.make_async_copy(src.at[pl.ds(0, _K)], sb[b],
                                  isem[b]).wait()
            pltpu.async_copy(h_tbl.at[sb[b]], msg[b], gsem[b])
            pltpu.async_copy(dst.at[pl.ds(ebase + i * _K, _K)], db[b],
                             gsem[b])
            pltpu.async_copy(attr.at[pl.ds(ebase + i * _K, _K)], ab[b],
                             gsem[b])

        for b in range(3):
            load_idx(b, b)
        start(0, 0)
        start(1, 1)

        zv = jnp.zeros((16,), jnp.float32)

        def zrow(e, carry):
            for j in range(_H // 16):
                msg[2][e, pl.ds(16 * j, 16)] = zv
            return carry

        lax.fori_loop(0, _K, zrow, 0)
        rbase = s * _RPT
        nfull = _RPT // _K
        for i in range(nfull):
            pltpu.sync_copy(msg[2], acc.at[pl.ds(rbase + i * _K, _K)])
        rem = _RPT - nfull * _K
        if rem:
            pltpu.sync_copy(msg[2].at[pl.ds(0, rem)],
                            acc.at[pl.ds(rbase + nfull * _K, rem)])

        @pl.when(s == _NS - 1)
        def _():
            pltpu.sync_copy(msg[2].at[pl.ds(0, _TAIL)],
                            acc.at[pl.ds(_RPT * _NS, _TAIL)])

        plsc.subcore_barrier()

        def compute(b):
            pltpu.make_async_copy(h_tbl.at[pl.ds(0, _K)], msg[b],
                                  gsem[b]).wait()
            pltpu.make_async_copy(dst.at[pl.ds(0, _K)], db[b],
                                  gsem[b]).wait()
            pltpu.make_async_copy(attr.at[pl.ds(0, _K)], ab[b],
                                  gsem[b]).wait()

            def grp(g, c2):
                av = ab[b][pl.ds(g * 16, 16)]
                for l in range(16):
                    a = jax.lax.gather(
                        av, jnp.full((16, 1), l, jnp.int32), _GDN, (1,),
                        mode=jax.lax.GatherScatterMode.PROMISE_IN_BOUNDS)
                    e = g * 16 + l
                    for j in range(_H // 16):
                        sl = pl.ds(16 * j, 16)
                        msg[b][e, sl] = msg[b][e, sl] * a
                return c2

            lax.fori_loop(0, _K // 16, grp, 0)
            pltpu.async_copy(msg[b], acc.at[db[b]], ssem[b], add=True)

        def wait_scatter(b):
            pltpu.make_async_copy(msg[b], acc.at[db[b]], ssem[b]).wait()

        def round_(r, carry):
            for b in range(3):
                i = r * 3 + b
                cur = b
                nxt2 = (b + 2) % 3

                @pl.when(i < _NCHUNK)
                def _():
                    compute(cur)

                @pl.when((i >= 1) & (i + 2 < _NCHUNK))
                def _():
                    wait_scatter(nxt2)

                @pl.when(i + 2 < _NCHUNK)
                def _():
                    start(i + 2, nxt2)

                @pl.when(i + 3 < _NCHUNK)
                def _():
                    load_idx(i + 3, cur)
            return carry

        lax.fori_loop(0, (_NCHUNK + 2) // 3, round_, 0)
        for cc in (_NCHUNK - 3, _NCHUNK - 2, _NCHUNK - 1):
            wait_scatter(cc % 3)
        plsc.subcore_barrier()
        pltpu.sync_copy(acc.at[pl.ds(rbase, _RPT)],
                        out_tbl.at[pl.ds(rbase, _RPT)])

        @pl.when(s == _NS - 1)
        def _():
            pltpu.sync_copy(acc.at[pl.ds(_RPT * _NS, _TAIL)],
                            out_tbl.at[pl.ds(_RPT * _NS, _TAIL)])

    @pl.when(c == 0)
    def _():
        run(h_lo, out_lo)

    @pl.when(c == 1)
    def _():
        run(h_hi, out_hi)


_R = 1000


def _dense_in(x, Win, b_in):
    def body(x_ref, w_ref, b_ref, h0_ref, lo_ref, hi_ref):
        h = jnp.maximum(_dot(x_ref[...], w_ref[...]) + b_ref[...], 0.0)
        h0_ref[...] = h
        lo_ref[...] = h[:, :_H]
        hi_ref[...] = h[:, _H:]

    return pl.pallas_call(
        body,
        grid=(_N // _R,),
        in_specs=[pl.BlockSpec((_R, _D), lambda i: (i, 0)),
                  pl.BlockSpec((_D, _D), lambda i: (0, 0)),
                  pl.BlockSpec((1, _D), lambda i: (0, 0))],
        out_specs=[pl.BlockSpec((_R, _D), lambda i: (i, 0)),
                   pl.BlockSpec((_R, _H), lambda i: (i, 0)),
                   pl.BlockSpec((_R, _H), lambda i: (i, 0))],
        out_shape=[jax.ShapeDtypeStruct((_N, _D), jnp.float32),
                   jax.ShapeDtypeStruct((_N, _H), jnp.float32),
                   jax.ShapeDtypeStruct((_N, _H), jnp.float32)],
    )(x, Win, b_in.reshape(1, _D))


def _mix_mm(agg_lo, agg_hi, h0, W, bc):
    def body(lo_ref, hi_ref, h0_ref, w_ref, t2_ref, s1_ref, s2_ref):
        i = pl.program_id(0)
        t = ((1.0 - _ALPHA)
             * jnp.concatenate([lo_ref[...], hi_ref[...]], axis=1)
             + _ALPHA * h0_ref[...])
        t2 = (1.0 - bc) * t + bc * _dot(t, w_ref[...])
        t2_ref[...] = t2

        @pl.when(i == 0)
        def _():
            s1_ref[...] = jnp.zeros_like(s1_ref)
            s2_ref[...] = jnp.zeros_like(s2_ref)

        s1_ref[...] += jnp.sum(t2, axis=0, keepdims=True)
        s2_ref[...] += jnp.sum(t2 * t2, axis=0, keepdims=True)

    return pl.pallas_call(
        body,
        grid=(_N // _R,),
        in_specs=[pl.BlockSpec((_R, _H), lambda i: (i, 0)),
                  pl.BlockSpec((_R, _H), lambda i: (i, 0)),
                  pl.BlockSpec((_R, _D), lambda i: (i, 0)),
                  pl.BlockSpec((_D, _D), lambda i: (0, 0))],
        out_specs=[pl.BlockSpec((_R, _D), lambda i: (i, 0)),
                   pl.BlockSpec((1, _D), lambda i: (0, 0)),
                   pl.BlockSpec((1, _D), lambda i: (0, 0))],
        out_shape=[jax.ShapeDtypeStruct((_N, _D), jnp.float32),
                   jax.ShapeDtypeStruct((1, _D), jnp.float32),
                   jax.ShapeDtypeStruct((1, _D), jnp.float32)],
    )(agg_lo, agg_hi, h0, W)


def _bn_relu(t2, s1, s2, g, b):
    def body(t2_ref, s1_ref, s2_ref, g_ref, b_ref, lo_ref, hi_ref):
        mu = s1_ref[...] * (1.0 / _N)
        var = s2_ref[...] * (1.0 / _N) - mu * mu
        scale = jax.lax.rsqrt(var + _EPS) * g_ref[...]
        h = jnp.maximum((t2_ref[...] - mu) * scale + b_ref[...], 0.0)
        lo_ref[...] = h[:, :_H]
        hi_ref[...] = h[:, _H:]

    return pl.pallas_call(
        body,
        grid=(_N // _R,),
        in_specs=[pl.BlockSpec((_R, _D), lambda i: (i, 0)),
                  pl.BlockSpec((1, _D), lambda i: (0, 0)),
                  pl.BlockSpec((1, _D), lambda i: (0, 0)),
                  pl.BlockSpec((1, _D), lambda i: (0, 0)),
                  pl.BlockSpec((1, _D), lambda i: (0, 0))],
        out_specs=[pl.BlockSpec((_R, _H), lambda i: (i, 0)),
                   pl.BlockSpec((_R, _H), lambda i: (i, 0))],
        out_shape=[jax.ShapeDtypeStruct((_N, _H), jnp.float32),
                   jax.ShapeDtypeStruct((_N, _H), jnp.float32)],
    )(t2, s1, s2, g, b)


def _dense_out(lo, hi, Wout, b_out):
    def body(lo_ref, hi_ref, w_ref, b_ref, o_ref):
        h = jnp.concatenate([lo_ref[...], hi_ref[...]], axis=1)
        o_ref[...] = _dot(h, w_ref[...]) + b_ref[...]

    return pl.pallas_call(
        body,
        grid=(_N // _R,),
        in_specs=[pl.BlockSpec((_R, _H), lambda i: (i, 0)),
                  pl.BlockSpec((_R, _H), lambda i: (i, 0)),
                  pl.BlockSpec((_D, _D), lambda i: (0, 0)),
                  pl.BlockSpec((1, _D), lambda i: (0, 0))],
        out_specs=pl.BlockSpec((_R, _D), lambda i: (i, 0)),
        out_shape=jax.ShapeDtypeStruct((_N, _D), jnp.float32),
    )(lo, hi, Wout, b_out.reshape(1, _D))


def kernel(x, edge_attr, Win, b_in, W1, gamma, beta, Wout, b_out, edge_index):
    src = edge_index[0]
    dst = edge_index[1]
    h0, lo, hi = _dense_in(x, Win, b_in)
    for l in range(_LAYERS):
        bc = float(np.log(_THETA / (l + 1) + 1.0))
        agg_lo, agg_hi = _sc_agg(lo, hi, src, dst, edge_attr)
        t2, s1, s2 = _mix_mm(agg_lo, agg_hi, h0, W1[l], bc)
        lo, hi = _bn_relu(t2, s1, s2,
                          gamma[l].reshape(1, _D), beta[l].reshape(1, _D))
    return _dense_out(lo, hi, Wout, b_out)


# speedup vs baseline: 1.0747x; 1.0747x over previous
import functools

import numpy as np
import jax
import jax.numpy as jnp
from jax import lax
from jax.experimental import pallas as pl
from jax.experimental.pallas import tpu as pltpu
from jax.experimental.pallas import tpu_sc as plsc

_ALPHA = 0.1
_THETA = 0.5
_LAYERS = 4
_EPS = 1e-5
_N, _E, _D = 10000, 160000, 256
_H = _D // 2
_NS = 16
_EPT = _E // _NS
_K = 80
_NCHUNK = _EPT // _K
_RPT = 624
_TAIL = _N - _RPT * _NS

_HIGH = jax.lax.Precision.HIGHEST
_GDN = jax.lax.GatherDimensionNumbers(
    offset_dims=(), collapsed_slice_dims=(0,), start_index_map=(0,))


def _dot(a, b):
    return jax.lax.dot_general(a, b, (((1,), (0,)), ((), ())),
                               precision=_HIGH,
                               preferred_element_type=jnp.float32)


@functools.partial(
    pl.kernel,
    out_type=[jax.ShapeDtypeStruct((_N, _H), jnp.float32),
              jax.ShapeDtypeStruct((_N, _H), jnp.float32)],
    mesh=plsc.VectorSubcoreMesh(core_axis_name="c", subcore_axis_name="s"),
    scratch_types=[
        [pltpu.VMEM((_K, _H), jnp.float32)] * 3,
        [pltpu.VMEM((_K,), jnp.int32)] * 3,
        [pltpu.VMEM((_K,), jnp.int32)] * 3,
        [pltpu.VMEM((_K,), jnp.float32)] * 3,
        pltpu.VMEM_SHARED((_N, _H), jnp.float32),
        [pltpu.SemaphoreType.DMA] * 3,
        [pltpu.SemaphoreType.DMA] * 3,
        [pltpu.SemaphoreType.DMA] * 3,
    ],
)
def _sc_agg(h_lo, h_hi, src, dst, attr, out_lo, out_hi,
            msg, sb, db, ab, acc, gsem, ssem, isem):
    c = lax.axis_index("c")
    s = lax.axis_index("s")

    def run(h_tbl, out_tbl):
        ebase = pl.multiple_of(s * _EPT, 8)

        def load_idx(i, b):
            pltpu.async_copy(src.at[pl.ds(ebase + i * _K, _K)], sb[b],
                             isem[b])

        def start(i, b):
            pltpu.make_async_copy(src.at[pl.ds(0, _K)], sb[b],
                                  isem[b]).wait()
            pltpu.async_copy(h_tbl.at[sb[b]], msg[b], gsem[b])
            pltpu.async_copy(dst.at[pl.ds(ebase + i * _K, _K)], db[b],
                             gsem[b])
            pltpu.async_copy(attr.at[pl.ds(ebase + i * _K, _K)], ab[b],
                             gsem[b])

        for b in range(3):
            load_idx(b, b)
        start(0, 0)
        start(1, 1)

        zv = jnp.zeros((16,), jnp.float32)

        def zrow(e, carry):
            for j in range(_H // 16):
                msg[2][e, pl.ds(16 * j, 16)] = zv
            return carry

        lax.fori_loop(0, _K, zrow, 0)
        rbase = s * _RPT
        nfull = _RPT // _K
        for i in range(nfull):
            pltpu.sync_copy(msg[2], acc.at[pl.ds(rbase + i * _K, _K)])
        rem = _RPT - nfull * _K
        if rem:
            pltpu.sync_copy(msg[2].at[pl.ds(0, rem)],
                            acc.at[pl.ds(rbase + nfull * _K, rem)])

        @pl.when(s == _NS - 1)
        def _():
            pltpu.sync_copy(msg[2].at[pl.ds(0, _TAIL)],
                            acc.at[pl.ds(_RPT * _NS, _TAIL)])

        plsc.subcore_barrier()

        def wait_in(b):
            pltpu.make_async_copy(h_tbl.at[pl.ds(0, _K)], msg[b],
                                  gsem[b]).wait()
            pltpu.make_async_copy(dst.at[pl.ds(0, _K)], db[b],
                                  gsem[b]).wait()
            pltpu.make_async_copy(attr.at[pl.ds(0, _K)], ab[b],
                                  gsem[b]).wait()

        def scale(b, glo, ghi):
            def grp(g, c2):
                av = ab[b][pl.ds(g * 16, 16)]
                for l in range(16):
                    a = jax.lax.gather(
                        av, jnp.full((16, 1), l, jnp.int32), _GDN, (1,),
                        mode=jax.lax.GatherScatterMode.PROMISE_IN_BOUNDS)
                    e = g * 16 + l
                    for j in range(_H // 16):
                        sl = pl.ds(16 * j, 16)
                        msg[b][e, sl] = msg[b][e, sl] * a
                return c2

            lax.fori_loop(glo, ghi, grp, 0)

        def wait_scatter(b):
            pltpu.make_async_copy(msg[b], acc.at[db[b]], ssem[b]).wait()

        def round_(r, carry):
            for b in range(3):
                i = r * 3 + b
                cur = b
                nxt2 = (b + 2) % 3

                @pl.when(i < _NCHUNK)
                def _():
                    wait_in(cur)
                    scale(cur, 0, 2)

                @pl.when((i >= 1) & (i + 2 < _NCHUNK))
                def _():
                    wait_scatter(nxt2)

                @pl.when(i + 2 < _NCHUNK)
                def _():
                    start(i + 2, nxt2)

                @pl.when(i + 3 < _NCHUNK)
                def _():
                    load_idx(i + 3, cur)

                @pl.when(i < _NCHUNK)
                def _():
                    scale(cur, 2, _K // 16)
                    pltpu.async_copy(msg[cur], acc.at[db[cur]], ssem[cur],
                                     add=True)
            return carry

        lax.fori_loop(0, (_NCHUNK + 2) // 3, round_, 0)
        for cc in (_NCHUNK - 3, _NCHUNK - 2, _NCHUNK - 1):
            wait_scatter(cc % 3)
        plsc.subcore_barrier()
        pltpu.sync_copy(acc.at[pl.ds(rbase, _RPT)],
                        out_tbl.at[pl.ds(rbase, _RPT)])

        @pl.when(s == _NS - 1)
        def _():
            pltpu.sync_copy(acc.at[pl.ds(_RPT * _NS, _TAIL)],
                            out_tbl.at[pl.ds(_RPT * _NS, _TAIL)])

    @pl.when(c == 0)
    def _():
        run(h_lo, out_lo)

    @pl.when(c == 1)
    def _():
        run(h_hi, out_hi)


_R = 2000


def _dense_in(x, Win, b_in):
    def body(x_ref, w_ref, b_ref, h0_ref, lo_ref, hi_ref):
        h = jnp.maximum(_dot(x_ref[...], w_ref[...]) + b_ref[...], 0.0)
        h0_ref[...] = h
        lo_ref[...] = h[:, :_H]
        hi_ref[...] = h[:, _H:]

    return pl.pallas_call(
        body,
        grid=(_N // _R,),
        in_specs=[pl.BlockSpec((_R, _D), lambda i: (i, 0)),
                  pl.BlockSpec((_D, _D), lambda i: (0, 0)),
                  pl.BlockSpec((1, _D), lambda i: (0, 0))],
        out_specs=[pl.BlockSpec((_R, _D), lambda i: (i, 0)),
                   pl.BlockSpec((_R, _H), lambda i: (i, 0)),
                   pl.BlockSpec((_R, _H), lambda i: (i, 0))],
        out_shape=[jax.ShapeDtypeStruct((_N, _D), jnp.float32),
                   jax.ShapeDtypeStruct((_N, _H), jnp.float32),
                   jax.ShapeDtypeStruct((_N, _H), jnp.float32)],
    )(x, Win, b_in.reshape(1, _D))


def _mix_mm(agg_lo, agg_hi, h0, W, bc):
    def body(lo_ref, hi_ref, h0_ref, w_ref, t2_ref, s1_ref, s2_ref):
        i = pl.program_id(0)
        t = ((1.0 - _ALPHA)
             * jnp.concatenate([lo_ref[...], hi_ref[...]], axis=1)
             + _ALPHA * h0_ref[...])
        t2 = (1.0 - bc) * t + bc * _dot(t, w_ref[...])
        t2_ref[...] = t2

        @pl.when(i == 0)
        def _():
            s1_ref[...] = jnp.zeros_like(s1_ref)
            s2_ref[...] = jnp.zeros_like(s2_ref)

        s1_ref[...] += jnp.sum(t2, axis=0, keepdims=True)
        s2_ref[...] += jnp.sum(t2 * t2, axis=0, keepdims=True)

    return pl.pallas_call(
        body,
        grid=(_N // _R,),
        in_specs=[pl.BlockSpec((_R, _H), lambda i: (i, 0)),
                  pl.BlockSpec((_R, _H), lambda i: (i, 0)),
                  pl.BlockSpec((_R, _D), lambda i: (i, 0)),
                  pl.BlockSpec((_D, _D), lambda i: (0, 0))],
        out_specs=[pl.BlockSpec((_R, _D), lambda i: (i, 0)),
                   pl.BlockSpec((1, _D), lambda i: (0, 0)),
                   pl.BlockSpec((1, _D), lambda i: (0, 0))],
        out_shape=[jax.ShapeDtypeStruct((_N, _D), jnp.float32),
                   jax.ShapeDtypeStruct((1, _D), jnp.float32),
                   jax.ShapeDtypeStruct((1, _D), jnp.float32)],
    )(agg_lo, agg_hi, h0, W)


def _bn_relu(t2, s1, s2, g, b):
    def body(t2_ref, s1_ref, s2_ref, g_ref, b_ref, lo_ref, hi_ref):
        mu = s1_ref[...] * (1.0 / _N)
        var = s2_ref[...] * (1.0 / _N) - mu * mu
        scale = jax.lax.rsqrt(var + _EPS) * g_ref[...]
        h = jnp.maximum((t2_ref[...] - mu) * scale + b_ref[...], 0.0)
        lo_ref[...] = h[:, :_H]
        hi_ref[...] = h[:, _H:]

    return pl.pallas_call(
        body,
        grid=(_N // _R,),
        in_specs=[pl.BlockSpec((_R, _D), lambda i: (i, 0)),
                  pl.BlockSpec((1, _D), lambda i: (0, 0)),
                  pl.BlockSpec((1, _D), lambda i: (0, 0)),
                  pl.BlockSpec((1, _D), lambda i: (0, 0)),
                  pl.BlockSpec((1, _D), lambda i: (0, 0))],
        out_specs=[pl.BlockSpec((_R, _H), lambda i: (i, 0)),
                   pl.BlockSpec((_R, _H), lambda i: (i, 0))],
        out_shape=[jax.ShapeDtypeStruct((_N, _H), jnp.float32),
                   jax.ShapeDtypeStruct((_N, _H), jnp.float32)],
    )(t2, s1, s2, g, b)


def _dense_out(lo, hi, Wout, b_out):
    def body(lo_ref, hi_ref, w_ref, b_ref, o_ref):
        h = jnp.concatenate([lo_ref[...], hi_ref[...]], axis=1)
        o_ref[...] = _dot(h, w_ref[...]) + b_ref[...]

    return pl.pallas_call(
        body,
        grid=(_N // _R,),
        in_specs=[pl.BlockSpec((_R, _H), lambda i: (i, 0)),
                  pl.BlockSpec((_R, _H), lambda i: (i, 0)),
                  pl.BlockSpec((_D, _D), lambda i: (0, 0)),
                  pl.BlockSpec((1, _D), lambda i: (0, 0))],
        out_specs=pl.BlockSpec((_R, _D), lambda i: (i, 0)),
        out_shape=jax.ShapeDtypeStruct((_N, _D), jnp.float32),
    )(lo, hi, Wout, b_out.reshape(1, _D))


def kernel(x, edge_attr, Win, b_in, W1, gamma, beta, Wout, b_out, edge_index):
    src = edge_index[0]
    dst = edge_index[1]
    h0, lo, hi = _dense_in(x, Win, b_in)
    for l in range(_LAYERS):
        bc = float(np.log(_THETA / (l + 1) + 1.0))
        agg_lo, agg_hi = _sc_agg(lo, hi, src, dst, edge_attr)
        t2, s1, s2 = _mix_mm(agg_lo, agg_hi, h0, W1[l], bc)
        lo, hi = _bn_relu(t2, s1, s2,
                          gamma[l].reshape(1, _D), beta[l].reshape(1, _D))
    return _dense_out(lo, hi, Wout, b_out)


# speedup vs baseline: 1.1071x; 1.0301x over previous
import functools

import numpy as np
import jax
import jax.numpy as jnp
from jax import lax
from jax.experimental import pallas as pl
from jax.experimental.pallas import tpu as pltpu
from jax.experimental.pallas import tpu_sc as plsc

_ALPHA = 0.1
_THETA = 0.5
_LAYERS = 4
_EPS = 1e-5
_N, _E, _D = 10000, 160000, 256
_H = _D // 2
_NS = 16
_EPT = _E // _NS
_K = 80
_NCHUNK = _EPT // _K
_RPT = 624
_TAIL = _N - _RPT * _NS

_HIGH = jax.lax.Precision.DEFAULT
_GDN = jax.lax.GatherDimensionNumbers(
    offset_dims=(), collapsed_slice_dims=(0,), start_index_map=(0,))


def _dot(a, b):
    return jax.lax.dot_general(a, b, (((1,), (0,)), ((), ())),
                               precision=_HIGH,
                               preferred_element_type=jnp.float32)


@functools.partial(
    pl.kernel,
    out_type=[jax.ShapeDtypeStruct((_N, _H), jnp.float32),
              jax.ShapeDtypeStruct((_N, _H), jnp.float32)],
    mesh=plsc.VectorSubcoreMesh(core_axis_name="c", subcore_axis_name="s"),
    scratch_types=[
        [pltpu.VMEM((_K, _H), jnp.float32)] * 3,
        [pltpu.VMEM((_K,), jnp.int32)] * 3,
        [pltpu.VMEM((_K,), jnp.int32)] * 3,
        [pltpu.VMEM((_K,), jnp.float32)] * 3,
        pltpu.VMEM_SHARED((_N, _H), jnp.float32),
        [pltpu.SemaphoreType.DMA] * 3,
        [pltpu.SemaphoreType.DMA] * 3,
        [pltpu.SemaphoreType.DMA] * 3,
    ],
)
def _sc_agg(h_lo, h_hi, src, dst, attr, out_lo, out_hi,
            msg, sb, db, ab, acc, gsem, ssem, isem):
    c = lax.axis_index("c")
    s = lax.axis_index("s")

    def run(h_tbl, out_tbl):
        ebase = pl.multiple_of(s * _EPT, 8)

        def load_idx(i, b):
            pltpu.async_copy(src.at[pl.ds(ebase + i * _K, _K)], sb[b],
                             isem[b])

        def start(i, b):
            pltpu.make_async_copy(src.at[pl.ds(0, _K)], sb[b],
                                  isem[b]).wait()
            pltpu.async_copy(h_tbl.at[sb[b]], msg[b], gsem[b])
            pltpu.async_copy(dst.at[pl.ds(ebase + i * _K, _K)], db[b],
                             gsem[b])
            pltpu.async_copy(attr.at[pl.ds(ebase + i * _K, _K)], ab[b],
                             gsem[b])

        for b in range(3):
            load_idx(b, b)
        start(0, 0)
        start(1, 1)

        zv = jnp.zeros((16,), jnp.float32)

        def zrow(e, carry):
            for j in range(_H // 16):
                msg[2][e, pl.ds(16 * j, 16)] = zv
            return carry

        lax.fori_loop(0, _K, zrow, 0)
        rbase = s * _RPT
        nfull = _RPT // _K
        for i in range(nfull):
            pltpu.sync_copy(msg[2], acc.at[pl.ds(rbase + i * _K, _K)])
        rem = _RPT - nfull * _K
        if rem:
            pltpu.sync_copy(msg[2].at[pl.ds(0, rem)],
                            acc.at[pl.ds(rbase + nfull * _K, rem)])

        @pl.when(s == _NS - 1)
        def _():
            pltpu.sync_copy(msg[2].at[pl.ds(0, _TAIL)],
                            acc.at[pl.ds(_RPT * _NS, _TAIL)])

        plsc.subcore_barrier()

        def wait_in(b):
            pltpu.make_async_copy(h_tbl.at[pl.ds(0, _K)], msg[b],
                                  gsem[b]).wait()
            pltpu.make_async_copy(dst.at[pl.ds(0, _K)], db[b],
                                  gsem[b]).wait()
            pltpu.make_async_copy(attr.at[pl.ds(0, _K)], ab[b],
                                  gsem[b]).wait()

        def scale(b, glo, ghi):
            def grp(g, c2):
                av = ab[b][pl.ds(g * 16, 16)]
                for l in range(16):
                    a = jax.lax.gather(
                        av, jnp.full((16, 1), l, jnp.int32), _GDN, (1,),
                        mode=jax.lax.GatherScatterMode.PROMISE_IN_BOUNDS)
                    e = g * 16 + l
                    for j in range(_H // 16):
                        sl = pl.ds(16 * j, 16)
                        msg[b][e, sl] = msg[b][e, sl] * a
                return c2

            lax.fori_loop(glo, ghi, grp, 0)

        def wait_scatter(b):
            pltpu.make_async_copy(msg[b], acc.at[db[b]], ssem[b]).wait()

        def round_(r, carry):
            for b in range(3):
                i = r * 3 + b
                cur = b
                nxt2 = (b + 2) % 3

                @pl.when(i < _NCHUNK)
                def _():
                    wait_in(cur)
                    scale(cur, 0, 2)

                @pl.when((i >= 1) & (i + 2 < _NCHUNK))
                def _():
                    wait_scatter(nxt2)

                @pl.when(i + 2 < _NCHUNK)
                def _():
                    start(i + 2, nxt2)

                @pl.when(i + 3 < _NCHUNK)
                def _():
                    load_idx(i + 3, cur)

                @pl.when(i < _NCHUNK)
                def _():
                    scale(cur, 2, _K // 16)
                    pltpu.async_copy(msg[cur], acc.at[db[cur]], ssem[cur],
                                     add=True)
            return carry

        lax.fori_loop(0, (_NCHUNK + 2) // 3, round_, 0)
        for cc in (_NCHUNK - 3, _NCHUNK - 2, _NCHUNK - 1):
            wait_scatter(cc % 3)
        plsc.subcore_barrier()
        pltpu.sync_copy(acc.at[pl.ds(rbase, _RPT)],
                        out_tbl.at[pl.ds(rbase, _RPT)])

        @pl.when(s == _NS - 1)
        def _():
            pltpu.sync_copy(acc.at[pl.ds(_RPT * _NS, _TAIL)],
                            out_tbl.at[pl.ds(_RPT * _NS, _TAIL)])

    @pl.when(c == 0)
    def _():
        run(h_lo, out_lo)

    @pl.when(c == 1)
    def _():
        run(h_hi, out_hi)


_R = 2000


def _dense_in(x, Win, b_in):
    def body(x_ref, w_ref, b_ref, h0_ref, lo_ref, hi_ref):
        h = jnp.maximum(_dot(x_ref[...], w_ref[...]) + b_ref[...], 0.0)
        h0_ref[...] = h
        lo_ref[...] = h[:, :_H]
        hi_ref[...] = h[:, _H:]

    return pl.pallas_call(
        body,
        grid=(_N // _R,),
        in_specs=[pl.BlockSpec((_R, _D), lambda i: (i, 0)),
                  pl.BlockSpec((_D, _D), lambda i: (0, 0)),
                  pl.BlockSpec((1, _D), lambda i: (0, 0))],
        out_specs=[pl.BlockSpec((_R, _D), lambda i: (i, 0)),
                   pl.BlockSpec((_R, _H), lambda i: (i, 0)),
                   pl.BlockSpec((_R, _H), lambda i: (i, 0))],
        out_shape=[jax.ShapeDtypeStruct((_N, _D), jnp.float32),
                   jax.ShapeDtypeStruct((_N, _H), jnp.float32),
                   jax.ShapeDtypeStruct((_N, _H), jnp.float32)],
    )(x, Win, b_in.reshape(1, _D))


def _mix_mm(agg_lo, agg_hi, h0, W, bc):
    def body(lo_ref, hi_ref, h0_ref, w_ref, t2_ref, s1_ref, s2_ref):
        i = pl.program_id(0)
        t = ((1.0 - _ALPHA)
             * jnp.concatenate([lo_ref[...], hi_ref[...]], axis=1)
             + _ALPHA * h0_ref[...])
        t2 = (1.0 - bc) * t + bc * _dot(t, w_ref[...])
        t2_ref[...] = t2

        @pl.when(i == 0)
        def _():
            s1_ref[...] = jnp.zeros_like(s1_ref)
            s2_ref[...] = jnp.zeros_like(s2_ref)

        s1_ref[...] += jnp.sum(t2, axis=0, keepdims=True)
        s2_ref[...] += jnp.sum(t2 * t2, axis=0, keepdims=True)

    return pl.pallas_call(
        body,
        grid=(_N // _R,),
        in_specs=[pl.BlockSpec((_R, _H), lambda i: (i, 0)),
                  pl.BlockSpec((_R, _H), lambda i: (i, 0)),
                  pl.BlockSpec((_R, _D), lambda i: (i, 0)),
                  pl.BlockSpec((_D, _D), lambda i: (0, 0))],
        out_specs=[pl.BlockSpec((_R, _D), lambda i: (i, 0)),
                   pl.BlockSpec((1, _D), lambda i: (0, 0)),
                   pl.BlockSpec((1, _D), lambda i: (0, 0))],
        out_shape=[jax.ShapeDtypeStruct((_N, _D), jnp.float32),
                   jax.ShapeDtypeStruct((1, _D), jnp.float32),
                   jax.ShapeDtypeStruct((1, _D), jnp.float32)],
    )(agg_lo, agg_hi, h0, W)


def _bn_relu(t2, s1, s2, g, b):
    def body(t2_ref, s1_ref, s2_ref, g_ref, b_ref, lo_ref, hi_ref):
        mu = s1_ref[...] * (1.0 / _N)
        var = s2_ref[...] * (1.0 / _N) - mu * mu
        scale = jax.lax.rsqrt(var + _EPS) * g_ref[...]
        h = jnp.maximum((t2_ref[...] - mu) * scale + b_ref[...], 0.0)
        lo_ref[...] = h[:, :_H]
        hi_ref[...] = h[:, _H:]

    return pl.pallas_call(
        body,
        grid=(_N // _R,),
        in_specs=[pl.BlockSpec((_R, _D), lambda i: (i, 0)),
                  pl.BlockSpec((1, _D), lambda i: (0, 0)),
                  pl.BlockSpec((1, _D), lambda i: (0, 0)),
                  pl.BlockSpec((1, _D), lambda i: (0, 0)),
                  pl.BlockSpec((1, _D), lambda i: (0, 0))],
        out_specs=[pl.BlockSpec((_R, _H), lambda i: (i, 0)),
                   pl.BlockSpec((_R, _H), lambda i: (i, 0))],
        out_shape=[jax.ShapeDtypeStruct((_N, _H), jnp.float32),
                   jax.ShapeDtypeStruct((_N, _H), jnp.float32)],
    )(t2, s1, s2, g, b)


def _dense_out(lo, hi, Wout, b_out):
    def body(lo_ref, hi_ref, w_ref, b_ref, o_ref):
        h = jnp.concatenate([lo_ref[...], hi_ref[...]], axis=1)
        o_ref[...] = _dot(h, w_ref[...]) + b_ref[...]

    return pl.pallas_call(
        body,
        grid=(_N // _R,),
        in_specs=[pl.BlockSpec((_R, _H), lambda i: (i, 0)),
                  pl.BlockSpec((_R, _H), lambda i: (i, 0)),
                  pl.BlockSpec((_D, _D), lambda i: (0, 0)),
                  pl.BlockSpec((1, _D), lambda i: (0, 0))],
        out_specs=pl.BlockSpec((_R, _D), lambda i: (i, 0)),
        out_shape=jax.ShapeDtypeStruct((_N, _D), jnp.float32),
    )(lo, hi, Wout, b_out.reshape(1, _D))


def kernel(x, edge_attr, Win, b_in, W1, gamma, beta, Wout, b_out, edge_index):
    src = edge_index[0]
    dst = edge_index[1]
    h0, lo, hi = _dense_in(x, Win, b_in)
    for l in range(_LAYERS):
        bc = float(np.log(_THETA / (l + 1) + 1.0))
        agg_lo, agg_hi = _sc_agg(lo, hi, src, dst, edge_attr)
        t2, s1, s2 = _mix_mm(agg_lo, agg_hi, h0, W1[l], bc)
        lo, hi = _bn_relu(t2, s1, s2,
                          gamma[l].reshape(1, _D), beta[l].reshape(1, _D))
    return _dense_out(lo, hi, Wout, b_out)
